# baseline (speedup 1.0000x reference)
import functools

import jax
import jax.numpy as jnp
from jax import lax
from jax.experimental import pallas as pl
from jax.experimental.pallas import tpu as pltpu
from jax.experimental.pallas import tpu_sc as plsc


NUM_CORES = 2
NUM_SUBCORES = 16
NUM_WORKERS = NUM_CORES * NUM_SUBCORES

GATHER_CHUNK = 128
MM_BLOCK_H = 2


def _sc_gather(table2, idx_flat):
    n = idx_flat.shape[0]
    d2 = table2.shape[1]
    b_per_w = n // NUM_WORKERS
    nchunk = b_per_w // GATHER_CHUNK
    assert n % NUM_WORKERS == 0 and b_per_w % GATHER_CHUNK == 0
    assert nchunk % 2 == 0

    mesh = plsc.VectorSubcoreMesh(core_axis_name="c", subcore_axis_name="s")

    @functools.partial(
        pl.kernel,
        out_type=jax.ShapeDtypeStruct((n, d2), table2.dtype),
        mesh=mesh,
        scratch_types=[
            pltpu.VMEM((b_per_w,), jnp.int32),
            pltpu.VMEM((GATHER_CHUNK, d2), jnp.float32),
            pltpu.VMEM((GATHER_CHUNK, d2), jnp.float32),
            pltpu.SemaphoreType.DMA,
            pltpu.SemaphoreType.DMA,
        ],
    )
    def gather_kernel(table_hbm, idx_hbm, out_hbm, idx_v, buf0, buf1, sem0, sem1):
        wid = lax.axis_index("s") * NUM_CORES + lax.axis_index("c")
        base = wid * b_per_w

        pltpu.sync_copy(idx_hbm.at[pl.ds(base, b_per_w)], idx_v)

        def start_gather(c, buf, sem):
            off = pl.multiple_of(c * GATHER_CHUNK, GATHER_CHUNK)
            for g in range(GATHER_CHUNK // 16):
                vec = idx_v[pl.ds(off + g * 16, 16)]
                for t in range(16):
                    row = lax.shift_right_logical(vec[t], 1)
                    pltpu.async_copy(
                        table_hbm.at[pl.ds(row, 1)],
                        buf.at[pl.ds(g * 16 + t, 1)],
                        sem,
                    )

        def wait_gather(buf, sem):
            pltpu.make_async_copy(
                table_hbm.at[pl.ds(0, GATHER_CHUNK)], buf, sem
            ).wait()

        def write_out(c, buf):
            row = base + c * GATHER_CHUNK
            pltpu.sync_copy(buf, out_hbm.at[pl.ds(row, GATHER_CHUNK)])

        start_gather(0, buf0, sem0)

        @pl.loop(0, nchunk // 2)
        def _(i):
            c0 = i * 2
            start_gather(c0 + 1, buf1, sem1)
            wait_gather(buf0, sem0)
            write_out(c0, buf0)

            @pl.when(c0 + 2 < nchunk)
            def _():
                start_gather(c0 + 2, buf0, sem0)

            wait_gather(buf1, sem1)
            write_out(c0 + 1, buf1)

    return gather_kernel(table2, idx_flat)


def _mm_body(emb_ref, par_ref, w_ref, out_ref):
    hb = out_ref.shape[0]
    b = out_ref.shape[1]
    m = out_ref.shape[2]
    d = w_ref.shape[1]
    lo = emb_ref[:, :d]
    hi = emb_ref[:, d:]
    par = par_ref[...]
    sel = lo + par * (hi - lo)
    acc = lax.dot_general(
        sel,
        w_ref[...],
        (((1,), (1,)), ((), ())),
        preferred_element_type=jnp.float32,
    )
    out_ref[...] = acc.reshape(hb, b, m)


def _tc_matmul(emb, par, W, b, h):
    n, d2 = emb.shape
    m, k = W.shape
    hb = MM_BLOCK_H
    grid = (h // hb,)
    return pl.pallas_call(
        _mm_body,
        grid=grid,
        in_specs=[
            pl.BlockSpec((hb * b, d2), lambda i: (i, 0)),
            pl.BlockSpec((hb * b, 1), lambda i: (i, 0)),
            pl.BlockSpec((m, k), lambda i: (0, 0)),
        ],
        out_specs=pl.BlockSpec((hb, b, m), lambda i: (i, 0, 0)),
        out_shape=jax.ShapeDtypeStruct((h, b, m), jnp.float32),
    )(emb, par, W)


def kernel(x, table, W):
    b, h = x.shape
    v, d = table.shape
    idx_flat = x.T.reshape(b * h).astype(jnp.int32)
    table2 = table.reshape(v // 2, 2 * d)
    par = (idx_flat & 1).astype(jnp.float32).reshape(b * h, 1)
    emb = _sc_gather(table2, idx_flat)
    out_hbm = _tc_matmul(emb, par, W, b, h)
    return jnp.transpose(out_hbm, (1, 0, 2))


# speedup vs baseline: 1.6773x; 1.6773x over previous
import functools

import jax
import jax.numpy as jnp
from jax import lax
from jax.experimental import pallas as pl
from jax.experimental.pallas import tpu as pltpu
from jax.experimental.pallas import tpu_sc as plsc


NUM_CORES = 2
NUM_SUBCORES = 16
NUM_WORKERS = NUM_CORES * NUM_SUBCORES

GATHER_CHUNK = 128
MM_BLOCK_H = 5


def _sc_gather(table, idx_flat):
    n = idx_flat.shape[0]
    d = table.shape[1]
    b_per_w = n // NUM_WORKERS
    nchunk = b_per_w // GATHER_CHUNK
    assert n % NUM_WORKERS == 0 and b_per_w % GATHER_CHUNK == 0
    assert nchunk % 2 == 0

    mesh = plsc.VectorSubcoreMesh(core_axis_name="c", subcore_axis_name="s")

    @functools.partial(
        pl.kernel,
        out_type=jax.ShapeDtypeStruct((n, d), table.dtype),
        mesh=mesh,
        scratch_types=[
            pltpu.VMEM((b_per_w,), jnp.int32),
            pltpu.VMEM((GATHER_CHUNK, d), jnp.float32),
            pltpu.VMEM((GATHER_CHUNK, d), jnp.float32),
            pltpu.SemaphoreType.DMA,
            pltpu.SemaphoreType.DMA,
        ],
    )
    def gather_kernel(table_hbm, idx_hbm, out_hbm, idx_v, buf0, buf1, sem0, sem1):
        wid = lax.axis_index("s") * NUM_CORES + lax.axis_index("c")
        base = wid * b_per_w

        pltpu.sync_copy(idx_hbm.at[pl.ds(base, b_per_w)], idx_v)

        def start_gather(c, buf, sem):
            off = pl.multiple_of(c * GATHER_CHUNK, GATHER_CHUNK)
            for g in range(GATHER_CHUNK // 16):
                vec = idx_v[pl.ds(off + g * 16, 16)]
                for t in range(16):
                    pltpu.async_copy(
                        table_hbm.at[pl.ds(vec[t], 1)],
                        buf.at[pl.ds(g * 16 + t, 1)],
                        sem,
                    )

        def wait_gather(buf, sem):
            pltpu.make_async_copy(
                table_hbm.at[pl.ds(0, GATHER_CHUNK)], buf, sem
            ).wait()

        def write_out(c, buf):
            row = base + c * GATHER_CHUNK
            pltpu.sync_copy(buf, out_hbm.at[pl.ds(row, GATHER_CHUNK)])

        start_gather(0, buf0, sem0)

        @pl.loop(0, nchunk // 2)
        def _(i):
            c0 = i * 2
            start_gather(c0 + 1, buf1, sem1)
            wait_gather(buf0, sem0)
            write_out(c0, buf0)

            @pl.when(c0 + 2 < nchunk)
            def _():
                start_gather(c0 + 2, buf0, sem0)

            wait_gather(buf1, sem1)
            write_out(c0 + 1, buf1)

    return gather_kernel(table, idx_flat)


def _mm_body(emb_ref, w_ref, out_ref):
    hb = out_ref.shape[0]
    b = out_ref.shape[1]
    m = out_ref.shape[2]
    acc = lax.dot_general(
        emb_ref[...],
        w_ref[...],
        (((1,), (1,)), ((), ())),
        preferred_element_type=jnp.float32,
    )
    out_ref[...] = acc.reshape(hb, b, m)


def _tc_matmul(emb, W, b, h):
    n, k = emb.shape
    m = W.shape[0]
    hb = MM_BLOCK_H
    grid = (h // hb,)
    return pl.pallas_call(
        _mm_body,
        grid=grid,
        in_specs=[
            pl.BlockSpec((hb * b, k), lambda i: (i, 0)),
            pl.BlockSpec((m, k), lambda i: (0, 0)),
        ],
        out_specs=pl.BlockSpec((hb, b, m), lambda i: (i, 0, 0)),
        out_shape=jax.ShapeDtypeStruct((h, b, m), jnp.float32),
    )(emb, W)


def kernel(x, table, W):
    b, h = x.shape
    idx_flat = x.T.reshape(b * h).astype(jnp.int32)
    emb = _sc_gather(table, idx_flat)
    out_hbm = _tc_matmul(emb, W, b, h)
    return jnp.transpose(out_hbm, (1, 0, 2))
